# baseline (speedup 1.0000x reference)
import jax
import jax.numpy as jnp
from jax.experimental import pallas as pl
from jax.experimental.pallas import tpu as pltpu

B, S, D = 1, 2048, 1024
E = 64
DFF = 2048
T = B * S

TG = 256
NG = T // TG
TM = 64
MAXTILES = T // TM + E


def _gate_body(x_ref, gw_ref, nw_ref, xn_ref, ids_ref, ps_ref, cnt_ref):
    xt = x_ref[...]
    eps = jnp.finfo(jnp.float32).eps
    var = jnp.mean(xt * xt, axis=-1, keepdims=True)
    xn = xt * jax.lax.rsqrt(var + eps) * nw_ref[...]
    xn_ref[...] = xn
    logits = jax.lax.dot_general(
        xn, gw_ref[...], (((1,), (1,)), ((), ())),
        preferred_element_type=jnp.float32)
    m = jnp.max(logits, axis=-1, keepdims=True)
    p = jnp.exp(logits - m)
    p = p / jnp.sum(p, axis=-1, keepdims=True)
    ids = jnp.argmax(logits, axis=-1).astype(jnp.int32)
    ids_ref[0, 0, :] = ids
    ps_ref[0, 0, :] = jnp.sum(p, axis=0)
    onehot = (jax.lax.broadcasted_iota(jnp.int32, (TG, E), 1)
              == ids[:, None]).astype(jnp.int32)
    cnt_ref[0, 0, :] = jnp.sum(onehot, axis=0)


def _gate(xf, gate_w, norm_w):
    return pl.pallas_call(
        _gate_body,
        grid=(NG,),
        in_specs=[
            pl.BlockSpec((TG, D), lambda k: (k, 0)),
            pl.BlockSpec((E, D), lambda k: (0, 0)),
            pl.BlockSpec((1, D), lambda k: (0, 0)),
        ],
        out_specs=[
            pl.BlockSpec((TG, D), lambda k: (k, 0)),
            pl.BlockSpec((1, 1, TG), lambda k: (k, 0, 0)),
            pl.BlockSpec((1, 1, E), lambda k: (k, 0, 0)),
            pl.BlockSpec((1, 1, E), lambda k: (k, 0, 0)),
        ],
        out_shape=[
            jax.ShapeDtypeStruct((T, D), jnp.float32),
            jax.ShapeDtypeStruct((NG, 1, TG), jnp.int32),
            jax.ShapeDtypeStruct((NG, 1, E), jnp.float32),
            jax.ShapeDtypeStruct((NG, 1, E), jnp.int32),
        ],
    )(xf, gate_w, norm_w.reshape(1, D))


def _sched_body(ids_ref, cnt_ref, ps_ref, owner_ref, nrows_ref, gidx_ref,
                lb_ref, tstart_ref, pado_ref, counts_ref, psum_ref):
    def init_e(e, _):
        counts_ref[e] = 0
        psum_ref[e] = 0.0
        return 0

    jax.lax.fori_loop(0, E, init_e, 0)
    for g in range(NG):
        def add_g(e, _, g=g):
            counts_ref[e] = counts_ref[e] + cnt_ref[g, 0, e]
            psum_ref[e] = psum_ref[e] + ps_ref[g, 0, e]
            return 0

        jax.lax.fori_loop(0, E, add_g, 0)

    def lb_step(e, acc):
        return acc + counts_ref[e].astype(jnp.float32) * psum_ref[e]

    lb_ref[0] = jax.lax.fori_loop(0, E, lb_step, 0.0)

    def estep(e, tot):
        c = counts_ref[e]
        tstart_ref[e] = tot
        pado_ref[e] = tot * TM
        return tot + (c + TM - 1) // TM

    total = jax.lax.fori_loop(0, E, estep, 0)

    def fill_e(e, _):
        c = counts_ref[e]
        ts = tstart_ref[e]

        def fill_k(j, _):
            owner_ref[ts + j] = e
            nrows_ref[ts + j] = jnp.minimum(c - j * TM, TM)
            return 0

        jax.lax.fori_loop(0, (c + TM - 1) // TM, fill_k, 0)
        return 0

    jax.lax.fori_loop(0, E, fill_e, 0)

    last = owner_ref[jnp.maximum(total - 1, 0)]

    def fill_tail(k, _):
        owner_ref[k] = last
        nrows_ref[k] = 0
        return 0

    jax.lax.fori_loop(total, MAXTILES, fill_tail, 0)

    for g in range(NG):
        def place(t, _, g=g):
            e = ids_ref[g, 0, t]
            p = pado_ref[e]
            gidx_ref[p] = g * TG + t
            pado_ref[e] = p + 1
            return 0

        jax.lax.fori_loop(0, TG, place, 0)


def _sched(ids3, cnt3, ps3):
    smem = pltpu.MemorySpace.SMEM
    return pl.pallas_call(
        _sched_body,
        in_specs=[
            pl.BlockSpec(memory_space=smem),
            pl.BlockSpec(memory_space=smem),
            pl.BlockSpec(memory_space=smem),
        ],
        out_specs=[
            pl.BlockSpec(memory_space=smem),
            pl.BlockSpec(memory_space=smem),
            pl.BlockSpec(memory_space=smem),
            pl.BlockSpec(memory_space=smem),
        ],
        out_shape=[
            jax.ShapeDtypeStruct((MAXTILES,), jnp.int32),
            jax.ShapeDtypeStruct((MAXTILES,), jnp.int32),
            jax.ShapeDtypeStruct((MAXTILES * TM,), jnp.int32),
            jax.ShapeDtypeStruct((1,), jnp.float32),
        ],
        scratch_shapes=[
            pltpu.SMEM((E,), jnp.int32),
            pltpu.SMEM((E,), jnp.int32),
            pltpu.SMEM((E,), jnp.int32),
            pltpu.SMEM((E,), jnp.float32),
        ],
    )(ids3, cnt3, ps3)


DFH = DFF // 2


def _ffn_body(owner_ref, nrows_ref, gidx_ref, xn_ref, w1a_ref, w1b_ref,
              w2a_ref, w2b_ref, out_ref, xs_ref, os_ref):
    k = pl.program_id(0)
    n = nrows_ref[k]

    @pl.when(n > 0)
    def _():
        base = k * TM

        def gather(r, c):
            idx = gidx_ref[base + r]
            xs_ref[pl.ds(r, 1), :] = xn_ref[pl.ds(idx, 1), :]
            return c
        jax.lax.fori_loop(0, n, gather, 0)

        xt = xs_ref[...]
        h1 = jax.lax.dot_general(
            xt, w1a_ref[0], (((1,), (1,)), ((), ())),
            preferred_element_type=jnp.float32)
        h1 = h1 * jax.nn.sigmoid(h1)
        o1 = jax.lax.dot_general(
            h1, w2a_ref[0], (((1,), (1,)), ((), ())),
            preferred_element_type=jnp.float32)
        h2 = jax.lax.dot_general(
            xt, w1b_ref[0], (((1,), (1,)), ((), ())),
            preferred_element_type=jnp.float32)
        h2 = h2 * jax.nn.sigmoid(h2)
        o2 = jax.lax.dot_general(
            h2, w2b_ref[0], (((1,), (1,)), ((), ())),
            preferred_element_type=jnp.float32)
        os_ref[...] = o1 + o2

        def scatter(r, c):
            idx = gidx_ref[base + r]
            out_ref[pl.ds(idx, 1), :] = os_ref[pl.ds(r, 1), :]
            return c
        jax.lax.fori_loop(0, n, scatter, 0)


def _ffn(xn, W1, W2, owner, nrows, gidx):
    grid_spec = pltpu.PrefetchScalarGridSpec(
        num_scalar_prefetch=3,
        grid=(MAXTILES,),
        in_specs=[
            pl.BlockSpec((T, D), lambda k, o, nr, g: (0, 0)),
            pl.BlockSpec((1, DFH, D), lambda k, o, nr, g: (o[k], 0, 0)),
            pl.BlockSpec((1, DFH, D), lambda k, o, nr, g: (o[k], 1, 0)),
            pl.BlockSpec((1, D, DFH), lambda k, o, nr, g: (o[k], 0, 0)),
            pl.BlockSpec((1, D, DFH), lambda k, o, nr, g: (o[k], 0, 1)),
        ],
        out_specs=pl.BlockSpec((T, D), lambda k, o, nr, g: (0, 0)),
        scratch_shapes=[
            pltpu.VMEM((TM, D), jnp.float32),
            pltpu.VMEM((TM, D), jnp.float32),
        ],
    )
    return pl.pallas_call(
        _ffn_body,
        grid_spec=grid_spec,
        out_shape=jax.ShapeDtypeStruct((T, D), jnp.float32),
    )(owner, nrows, gidx, xn, W1, W1, W2, W2)


def kernel(x, gate_w, W1, W2, norm_w, scale):
    xf = x.reshape(T, D)
    xn, ids3, ps3, cnt3 = _gate(xf, gate_w, norm_w)

    owner, nrows, gidx, lbp = _sched(ids3, cnt3, ps3)
    lb = lbp[0] * jnp.float32(E / (T * T))

    out = _ffn(xn, W1, W2, owner, nrows, gidx)

    s = jax.nn.sigmoid(scale)
    y = x + s * out.reshape(B, S, D)
    return y, lb


# speedup vs baseline: 1.0373x; 1.0373x over previous
import jax
import jax.numpy as jnp
from jax.experimental import pallas as pl
from jax.experimental.pallas import tpu as pltpu

B, S, D = 1, 2048, 1024
E = 64
DFF = 2048
T = B * S

TG = 256
NG = T // TG
TM = 64
MAXTILES = T // TM + E


def _gate_body(x_ref, gw_ref, nw_ref, xn_ref, ids_ref, ps_ref, cnt_ref):
    xt = x_ref[...]
    eps = jnp.finfo(jnp.float32).eps
    var = jnp.mean(xt * xt, axis=-1, keepdims=True)
    xn = xt * jax.lax.rsqrt(var + eps) * nw_ref[...]
    xn_ref[...] = xn
    logits = jax.lax.dot_general(
        xn, gw_ref[...], (((1,), (1,)), ((), ())),
        preferred_element_type=jnp.float32)
    m = jnp.max(logits, axis=-1, keepdims=True)
    p = jnp.exp(logits - m)
    p = p / jnp.sum(p, axis=-1, keepdims=True)
    ids = jnp.argmax(logits, axis=-1).astype(jnp.int32)
    ids_ref[0, 0, :] = ids

    @pl.when(pl.program_id(0) == 0)
    def _():
        ps_ref[...] = jnp.zeros_like(ps_ref)
        cnt_ref[...] = jnp.zeros_like(cnt_ref)

    ps_ref[0, 0, :] += jnp.sum(p, axis=0)
    onehot = (jax.lax.broadcasted_iota(jnp.int32, (TG, E), 1)
              == ids[:, None]).astype(jnp.int32)
    cnt_ref[0, 0, :] += jnp.sum(onehot, axis=0)


def _gate(xf, gate_w, norm_w):
    return pl.pallas_call(
        _gate_body,
        grid=(NG,),
        in_specs=[
            pl.BlockSpec((TG, D), lambda k: (k, 0)),
            pl.BlockSpec((E, D), lambda k: (0, 0)),
            pl.BlockSpec((1, D), lambda k: (0, 0)),
        ],
        out_specs=[
            pl.BlockSpec((TG, D), lambda k: (k, 0)),
            pl.BlockSpec((1, 1, TG), lambda k: (k, 0, 0)),
            pl.BlockSpec((1, 1, E), lambda k: (0, 0, 0)),
            pl.BlockSpec((1, 1, E), lambda k: (0, 0, 0)),
        ],
        out_shape=[
            jax.ShapeDtypeStruct((T, D), jnp.float32),
            jax.ShapeDtypeStruct((NG, 1, TG), jnp.int32),
            jax.ShapeDtypeStruct((1, 1, E), jnp.float32),
            jax.ShapeDtypeStruct((1, 1, E), jnp.int32),
        ],
    )(xf, gate_w, norm_w.reshape(1, D))


def _sched_body(ids_ref, counts_ref, owner_ref, nrows_ref, gidx_ref,
                tstart_ref, pado_ref):
    def estep(e, tot):
        c = counts_ref[e]
        tstart_ref[e] = tot
        pado_ref[e] = tot * TM
        return tot + (c + TM - 1) // TM

    total = jax.lax.fori_loop(0, E, estep, 0)

    def fill_e(e, _):
        c = counts_ref[e]
        ts = tstart_ref[e]

        def fill_k(j, _):
            owner_ref[ts + j] = e
            nrows_ref[ts + j] = jnp.minimum(c - j * TM, TM)
            return 0

        jax.lax.fori_loop(0, (c + TM - 1) // TM, fill_k, 0)
        return 0

    jax.lax.fori_loop(0, E, fill_e, 0)

    last = owner_ref[jnp.maximum(total - 1, 0)]

    def fill_tail(k, _):
        owner_ref[k] = last
        nrows_ref[k] = 0
        return 0

    jax.lax.fori_loop(total, MAXTILES, fill_tail, 0)

    def place(t, _):
        e = ids_ref[t]
        p = pado_ref[e]
        gidx_ref[p] = t
        pado_ref[e] = p + 1
        return 0

    jax.lax.fori_loop(0, T, place, 0)


def _sched(ids, counts):
    smem = pltpu.MemorySpace.SMEM
    return pl.pallas_call(
        _sched_body,
        in_specs=[
            pl.BlockSpec(memory_space=smem),
            pl.BlockSpec(memory_space=smem),
        ],
        out_specs=[
            pl.BlockSpec(memory_space=smem),
            pl.BlockSpec(memory_space=smem),
            pl.BlockSpec(memory_space=smem),
        ],
        out_shape=[
            jax.ShapeDtypeStruct((MAXTILES,), jnp.int32),
            jax.ShapeDtypeStruct((MAXTILES,), jnp.int32),
            jax.ShapeDtypeStruct((MAXTILES * TM,), jnp.int32),
        ],
        scratch_shapes=[
            pltpu.SMEM((E,), jnp.int32),
            pltpu.SMEM((E,), jnp.int32),
        ],
    )(ids, counts)


DFH = DFF // 2


def _ffn_body(owner_ref, nrows_ref, gidx_ref, xn_ref, w1a_ref, w1b_ref,
              w2a_ref, w2b_ref, out_ref, xs_ref, os_ref):
    k = pl.program_id(0)
    n = nrows_ref[k]

    @pl.when(n > 0)
    def _():
        base = k * TM

        def gather(r, c):
            idx = gidx_ref[base + r]
            xs_ref[pl.ds(r, 1), :] = xn_ref[pl.ds(idx, 1), :]
            return c
        jax.lax.fori_loop(0, n, gather, 0)

        xt = xs_ref[...]
        h1 = jax.lax.dot_general(
            xt, w1a_ref[0], (((1,), (1,)), ((), ())),
            preferred_element_type=jnp.float32)
        h1 = h1 * jax.nn.sigmoid(h1)
        o1 = jax.lax.dot_general(
            h1, w2a_ref[0], (((1,), (1,)), ((), ())),
            preferred_element_type=jnp.float32)
        h2 = jax.lax.dot_general(
            xt, w1b_ref[0], (((1,), (1,)), ((), ())),
            preferred_element_type=jnp.float32)
        h2 = h2 * jax.nn.sigmoid(h2)
        o2 = jax.lax.dot_general(
            h2, w2b_ref[0], (((1,), (1,)), ((), ())),
            preferred_element_type=jnp.float32)
        os_ref[...] = o1 + o2

        def scatter(r, c):
            idx = gidx_ref[base + r]
            out_ref[pl.ds(idx, 1), :] = os_ref[pl.ds(r, 1), :]
            return c
        jax.lax.fori_loop(0, n, scatter, 0)


def _ffn(xn, W1, W2, owner, nrows, gidx):
    grid_spec = pltpu.PrefetchScalarGridSpec(
        num_scalar_prefetch=3,
        grid=(MAXTILES,),
        in_specs=[
            pl.BlockSpec((T, D), lambda k, o, nr, g: (0, 0)),
            pl.BlockSpec((1, DFH, D), lambda k, o, nr, g: (o[k], 0, 0)),
            pl.BlockSpec((1, DFH, D), lambda k, o, nr, g: (o[k], 1, 0)),
            pl.BlockSpec((1, D, DFH), lambda k, o, nr, g: (o[k], 0, 0)),
            pl.BlockSpec((1, D, DFH), lambda k, o, nr, g: (o[k], 0, 1)),
        ],
        out_specs=pl.BlockSpec((T, D), lambda k, o, nr, g: (0, 0)),
        scratch_shapes=[
            pltpu.VMEM((TM, D), jnp.float32),
            pltpu.VMEM((TM, D), jnp.float32),
        ],
    )
    return pl.pallas_call(
        _ffn_body,
        grid_spec=grid_spec,
        out_shape=jax.ShapeDtypeStruct((T, D), jnp.float32),
    )(owner, nrows, gidx, xn, W1, W1, W2, W2)


def kernel(x, gate_w, W1, W2, norm_w, scale):
    xf = x.reshape(T, D)
    xn, ids3, ps3, cnt3 = _gate(xf, gate_w, norm_w)
    ids = ids3.reshape(T)
    probsum = ps3.reshape(E)
    counts = cnt3.reshape(E)

    lb = (E * jnp.sum((counts.astype(jnp.float32) / T) * (probsum / T))
          ).astype(jnp.float32)

    owner, nrows, gidx = _sched(ids, counts)

    out = _ffn(xn, W1, W2, owner, nrows, gidx)

    s = jax.nn.sigmoid(scale)
    y = x + s * out.reshape(B, S, D)
    return y, lb


# speedup vs baseline: 1.0404x; 1.0030x over previous
import jax
import jax.numpy as jnp
from jax.experimental import pallas as pl
from jax.experimental.pallas import tpu as pltpu

B, S, D = 1, 2048, 1024
E = 64
DFF = 2048
T = B * S

TG = 1024
NG = T // TG
TM = 64
MAXTILES = T // TM + E


def _gate_body(x_ref, gw_ref, nw_ref, xn_ref, ids_ref, ps_ref, cnt_ref):
    xt = x_ref[...]
    eps = jnp.finfo(jnp.float32).eps
    var = jnp.mean(xt * xt, axis=-1, keepdims=True)
    xn = xt * jax.lax.rsqrt(var + eps) * nw_ref[...]
    xn_ref[...] = xn
    logits = jax.lax.dot_general(
        xn, gw_ref[...], (((1,), (1,)), ((), ())),
        preferred_element_type=jnp.float32)
    m = jnp.max(logits, axis=-1, keepdims=True)
    p = jnp.exp(logits - m)
    p = p / jnp.sum(p, axis=-1, keepdims=True)
    ids = jnp.argmax(logits, axis=-1).astype(jnp.int32)
    ids_ref[0, 0, :] = ids

    @pl.when(pl.program_id(0) == 0)
    def _():
        ps_ref[...] = jnp.zeros_like(ps_ref)
        cnt_ref[...] = jnp.zeros_like(cnt_ref)

    ps_ref[0, 0, :] += jnp.sum(p, axis=0)
    onehot = (jax.lax.broadcasted_iota(jnp.int32, (TG, E), 1)
              == ids[:, None]).astype(jnp.int32)
    cnt_ref[0, 0, :] += jnp.sum(onehot, axis=0)


def _gate(xf, gate_w, norm_w):
    return pl.pallas_call(
        _gate_body,
        grid=(NG,),
        in_specs=[
            pl.BlockSpec((TG, D), lambda k: (k, 0)),
            pl.BlockSpec((E, D), lambda k: (0, 0)),
            pl.BlockSpec((1, D), lambda k: (0, 0)),
        ],
        out_specs=[
            pl.BlockSpec((TG, D), lambda k: (k, 0)),
            pl.BlockSpec((1, 1, TG), lambda k: (k, 0, 0)),
            pl.BlockSpec((1, 1, E), lambda k: (0, 0, 0)),
            pl.BlockSpec((1, 1, E), lambda k: (0, 0, 0)),
        ],
        out_shape=[
            jax.ShapeDtypeStruct((T, D), jnp.float32),
            jax.ShapeDtypeStruct((NG, 1, TG), jnp.int32),
            jax.ShapeDtypeStruct((1, 1, E), jnp.float32),
            jax.ShapeDtypeStruct((1, 1, E), jnp.int32),
        ],
    )(xf, gate_w, norm_w.reshape(1, D))


def _sched_body(ids_ref, counts_ref, owner_ref, nrows_ref, gidx_ref,
                tstart_ref, pado_ref):
    def estep(e, tot):
        c = counts_ref[e]
        tstart_ref[e] = tot
        pado_ref[e] = tot * TM
        return tot + (c + TM - 1) // TM

    total = jax.lax.fori_loop(0, E, estep, 0)

    def fill_e(e, _):
        c = counts_ref[e]
        ts = tstart_ref[e]

        def fill_k(j, _):
            owner_ref[ts + j] = e
            nrows_ref[ts + j] = jnp.minimum(c - j * TM, TM)
            return 0

        jax.lax.fori_loop(0, (c + TM - 1) // TM, fill_k, 0)
        return 0

    jax.lax.fori_loop(0, E, fill_e, 0)

    last = owner_ref[jnp.maximum(total - 1, 0)]

    def fill_tail(k, _):
        owner_ref[k] = last
        nrows_ref[k] = 0
        return 0

    jax.lax.fori_loop(total, MAXTILES, fill_tail, 0)

    def place(t, _):
        e = ids_ref[t]
        p = pado_ref[e]
        gidx_ref[p] = t
        pado_ref[e] = p + 1
        return 0

    jax.lax.fori_loop(0, T, place, 0)


def _sched(ids, counts):
    smem = pltpu.MemorySpace.SMEM
    return pl.pallas_call(
        _sched_body,
        in_specs=[
            pl.BlockSpec(memory_space=smem),
            pl.BlockSpec(memory_space=smem),
        ],
        out_specs=[
            pl.BlockSpec(memory_space=smem),
            pl.BlockSpec(memory_space=smem),
            pl.BlockSpec(memory_space=smem),
        ],
        out_shape=[
            jax.ShapeDtypeStruct((MAXTILES,), jnp.int32),
            jax.ShapeDtypeStruct((MAXTILES,), jnp.int32),
            jax.ShapeDtypeStruct((MAXTILES * TM,), jnp.int32),
        ],
        scratch_shapes=[
            pltpu.SMEM((E,), jnp.int32),
            pltpu.SMEM((E,), jnp.int32),
        ],
    )(ids, counts)


DFH = DFF // 2


def _ffn_body(owner_ref, nrows_ref, gidx_ref, xn_ref, w1a_ref, w1b_ref,
              w2a_ref, w2b_ref, out_ref, xs_ref, os_ref):
    k = pl.program_id(0)
    n = nrows_ref[k]

    @pl.when(n > 0)
    def _():
        base = k * TM

        def gather(r, c):
            idx = gidx_ref[base + r]
            xs_ref[pl.ds(r, 1), :] = xn_ref[pl.ds(idx, 1), :]
            return c
        jax.lax.fori_loop(0, n, gather, 0)

        xt = xs_ref[...]
        h1 = jax.lax.dot_general(
            xt, w1a_ref[0], (((1,), (1,)), ((), ())),
            preferred_element_type=jnp.float32)
        h1 = h1 * jax.nn.sigmoid(h1)
        o1 = jax.lax.dot_general(
            h1, w2a_ref[0], (((1,), (1,)), ((), ())),
            preferred_element_type=jnp.float32)
        h2 = jax.lax.dot_general(
            xt, w1b_ref[0], (((1,), (1,)), ((), ())),
            preferred_element_type=jnp.float32)
        h2 = h2 * jax.nn.sigmoid(h2)
        o2 = jax.lax.dot_general(
            h2, w2b_ref[0], (((1,), (1,)), ((), ())),
            preferred_element_type=jnp.float32)
        os_ref[...] = o1 + o2

        def scatter(r, c):
            idx = gidx_ref[base + r]
            out_ref[pl.ds(idx, 1), :] = os_ref[pl.ds(r, 1), :]
            return c
        jax.lax.fori_loop(0, n, scatter, 0)


def _ffn(xn, W1, W2, owner, nrows, gidx):
    grid_spec = pltpu.PrefetchScalarGridSpec(
        num_scalar_prefetch=3,
        grid=(MAXTILES,),
        in_specs=[
            pl.BlockSpec((T, D), lambda k, o, nr, g: (0, 0)),
            pl.BlockSpec((1, DFH, D), lambda k, o, nr, g: (o[k], 0, 0)),
            pl.BlockSpec((1, DFH, D), lambda k, o, nr, g: (o[k], 1, 0)),
            pl.BlockSpec((1, D, DFH), lambda k, o, nr, g: (o[k], 0, 0)),
            pl.BlockSpec((1, D, DFH), lambda k, o, nr, g: (o[k], 0, 1)),
        ],
        out_specs=pl.BlockSpec((T, D), lambda k, o, nr, g: (0, 0)),
        scratch_shapes=[
            pltpu.VMEM((TM, D), jnp.float32),
            pltpu.VMEM((TM, D), jnp.float32),
        ],
    )
    return pl.pallas_call(
        _ffn_body,
        grid_spec=grid_spec,
        out_shape=jax.ShapeDtypeStruct((T, D), jnp.float32),
    )(owner, nrows, gidx, xn, W1, W1, W2, W2)


def kernel(x, gate_w, W1, W2, norm_w, scale):
    xf = x.reshape(T, D)
    xn, ids3, ps3, cnt3 = _gate(xf, gate_w, norm_w)
    ids = ids3.reshape(T)
    probsum = ps3.reshape(E)
    counts = cnt3.reshape(E)

    lb = (E * jnp.sum((counts.astype(jnp.float32) / T) * (probsum / T))
          ).astype(jnp.float32)

    owner, nrows, gidx = _sched(ids, counts)

    out = _ffn(xn, W1, W2, owner, nrows, gidx)

    s = jax.nn.sigmoid(scale)
    y = x + s * out.reshape(B, S, D)
    return y, lb
